# baseline (speedup 1.0000x reference)
import jax
import jax.numpy as jnp
from jax.experimental import pallas as pl
from jax.experimental.pallas import tpu as pltpu

_RANK = 1024
_N_OUT = 8
_N_PROC = 32
_K = 3
_BLK = 1024


def _router_kernel(x_ref, wo_ref, wp_ref, ow_ref, pi_ref):
    xb = x_ref[...]
    dn = (((1,), (1,)), ((), ()))
    so = jax.lax.dot_general(wo_ref[...], xb, dn,
                             preferred_element_type=jnp.float32)
    sp = jax.lax.dot_general(wp_ref[...], xb, dn,
                             preferred_element_type=jnp.float32)

    m = jnp.max(so, axis=0, keepdims=True)
    e = jnp.exp(so - m)
    ow = e / jnp.sum(e, axis=0, keepdims=True)
    ow_ref[...] = ow.T

    iota = jax.lax.broadcasted_iota(jnp.int32, (_N_PROC, _BLK), 0)
    s = sp
    idxs = []
    for _ in range(_K):
        mx = jnp.max(s, axis=0, keepdims=True)
        idx = jnp.min(jnp.where(s >= mx, iota, _N_PROC), axis=0, keepdims=True)
        idxs.append(idx)
        s = jnp.where(iota == idx, -jnp.inf, s)
    pi = jnp.concatenate(idxs, axis=0)
    pi_ref[...] = pi.T


@jax.jit
def kernel(x, W_out, W_proc):
    B, S, R = x.shape
    n_tok = B * S
    xf = x.reshape(n_tok, R)
    grid = (n_tok // _BLK,)
    ow, pi = pl.pallas_call(
        _router_kernel,
        grid=grid,
        in_specs=[
            pl.BlockSpec((_BLK, R), lambda i: (i, 0)),
            pl.BlockSpec((_N_OUT, R), lambda i: (0, 0)),
            pl.BlockSpec((_N_PROC, R), lambda i: (0, 0)),
        ],
        out_specs=[
            pl.BlockSpec((_BLK, _N_OUT), lambda i: (i, 0)),
            pl.BlockSpec((_BLK, _K), lambda i: (i, 0)),
        ],
        out_shape=[
            jax.ShapeDtypeStruct((n_tok, _N_OUT), jnp.float32),
            jax.ShapeDtypeStruct((n_tok, _K), jnp.int32),
        ],
        compiler_params=pltpu.CompilerParams(
            dimension_semantics=("arbitrary",),
        ),
    )(xf, W_out, W_proc)
    return ow.reshape(B, S, _N_OUT), pi.reshape(B, S, _K)


# speedup vs baseline: 1.5995x; 1.5995x over previous
import jax
import jax.numpy as jnp
from jax.experimental import pallas as pl
from jax.experimental.pallas import tpu as pltpu

_RANK = 1024
_N_OUT = 8
_N_PROC = 32
_K = 3
_BLK = 2048


def _router_kernel(x_ref, wo_ref, wp_ref, ow_ref, pi_ref):
    xb = x_ref[...]
    dn = (((1,), (1,)), ((), ()))
    so = jax.lax.dot_general(wo_ref[...], xb, dn,
                             preferred_element_type=jnp.float32)
    sp = jax.lax.dot_general(wp_ref[...], xb, dn,
                             preferred_element_type=jnp.float32)

    m = jnp.max(so, axis=0, keepdims=True)
    e = jnp.exp(so - m)
    ow_ref[...] = e / jnp.sum(e, axis=0, keepdims=True)

    iota = jax.lax.broadcasted_iota(jnp.int32, (_N_PROC, _BLK), 0)
    s = sp
    for j in range(_K):
        mx = jnp.max(s, axis=0, keepdims=True)
        idx = jnp.min(jnp.where(s >= mx, iota, _N_PROC), axis=0, keepdims=True)
        pi_ref[j:j + 1, :] = idx
        s = jnp.where(iota == idx, -jnp.inf, s)


@jax.jit
def kernel(x, W_out, W_proc):
    B, S, R = x.shape
    n_tok = B * S
    xf = x.reshape(n_tok, R)
    grid = (n_tok // _BLK,)
    ow_t, pi_t = pl.pallas_call(
        _router_kernel,
        grid=grid,
        in_specs=[
            pl.BlockSpec((_BLK, R), lambda i: (i, 0)),
            pl.BlockSpec((_N_OUT, R), lambda i: (0, 0)),
            pl.BlockSpec((_N_PROC, R), lambda i: (0, 0)),
        ],
        out_specs=[
            pl.BlockSpec((_N_OUT, _BLK), lambda i: (0, i)),
            pl.BlockSpec((_K, _BLK), lambda i: (0, i)),
        ],
        out_shape=[
            jax.ShapeDtypeStruct((_N_OUT, n_tok), jnp.float32),
            jax.ShapeDtypeStruct((_K, n_tok), jnp.int32),
        ],
        compiler_params=pltpu.CompilerParams(
            dimension_semantics=("arbitrary",),
        ),
    )(xf, W_out, W_proc)
    ow = ow_t.T.reshape(B, S, _N_OUT)
    pi = pi_t.T.reshape(B, S, _K)
    return ow, pi


# speedup vs baseline: 1.6315x; 1.0200x over previous
import jax
import jax.numpy as jnp
from jax.experimental import pallas as pl
from jax.experimental.pallas import tpu as pltpu

_RANK = 1024
_N_OUT = 8
_N_PROC = 32
_K = 3
_BLK = 4096


def _router_kernel(x_ref, wo_ref, wp_ref, ow_ref, pi_ref):
    xb = x_ref[...]
    dn = (((1,), (1,)), ((), ()))
    so = jax.lax.dot_general(wo_ref[...], xb, dn,
                             preferred_element_type=jnp.float32)
    sp = jax.lax.dot_general(wp_ref[...], xb, dn,
                             preferred_element_type=jnp.float32)

    m = jnp.max(so, axis=0, keepdims=True)
    e = jnp.exp(so - m)
    ow_ref[...] = e / jnp.sum(e, axis=0, keepdims=True)

    iota = jax.lax.broadcasted_iota(jnp.int32, (_N_PROC, _BLK), 0)
    s = sp
    for j in range(_K):
        mx = jnp.max(s, axis=0, keepdims=True)
        idx = jnp.min(jnp.where(s >= mx, iota, _N_PROC), axis=0, keepdims=True)
        pi_ref[j:j + 1, :] = idx
        s = jnp.where(iota == idx, -jnp.inf, s)


@jax.jit
def kernel(x, W_out, W_proc):
    B, S, R = x.shape
    n_tok = B * S
    xf = x.reshape(n_tok, R)
    grid = (n_tok // _BLK,)
    ow_t, pi_t = pl.pallas_call(
        _router_kernel,
        grid=grid,
        in_specs=[
            pl.BlockSpec((_BLK, R), lambda i: (i, 0)),
            pl.BlockSpec((_N_OUT, R), lambda i: (0, 0)),
            pl.BlockSpec((_N_PROC, R), lambda i: (0, 0)),
        ],
        out_specs=[
            pl.BlockSpec((_N_OUT, _BLK), lambda i: (0, i)),
            pl.BlockSpec((_K, _BLK), lambda i: (0, i)),
        ],
        out_shape=[
            jax.ShapeDtypeStruct((_N_OUT, n_tok), jnp.float32),
            jax.ShapeDtypeStruct((_K, n_tok), jnp.int32),
        ],
        compiler_params=pltpu.CompilerParams(
            dimension_semantics=("arbitrary",),
        ),
    )(xf, W_out, W_proc)
    ow = ow_t.T.reshape(B, S, _N_OUT)
    pi = pi_t.T.reshape(B, S, _K)
    return ow, pi


# speedup vs baseline: 1.6547x; 1.0142x over previous
import jax
import jax.numpy as jnp
from jax.experimental import pallas as pl
from jax.experimental.pallas import tpu as pltpu

_RANK = 1024
_HALF = _RANK // 2
_N_OUT = 8
_N_PROC = 32
_K = 3
_BLK = 4096


def _router_kernel(x1_ref, x2_ref, wo_ref, wp_ref, ow_ref, pi_ref):
    x1 = x1_ref[...]
    x2 = x2_ref[...]
    wo = wo_ref[...]
    wp = wp_ref[...]
    dn = (((1,), (1,)), ((), ()))
    so = (jax.lax.dot_general(wo[:, :_HALF], x1, dn,
                              preferred_element_type=jnp.float32)
          + jax.lax.dot_general(wo[:, _HALF:], x2, dn,
                                preferred_element_type=jnp.float32))
    sp = (jax.lax.dot_general(wp[:, :_HALF], x1, dn,
                              preferred_element_type=jnp.float32)
          + jax.lax.dot_general(wp[:, _HALF:], x2, dn,
                                preferred_element_type=jnp.float32))

    m = jnp.max(so, axis=0, keepdims=True)
    e = jnp.exp(so - m)
    ow_ref[...] = e / jnp.sum(e, axis=0, keepdims=True)

    iota = jax.lax.broadcasted_iota(jnp.int32, (_N_PROC, _BLK), 0)
    s = sp
    for j in range(_K):
        mx = jnp.max(s, axis=0, keepdims=True)
        idx = jnp.min(jnp.where(s >= mx, iota, _N_PROC), axis=0, keepdims=True)
        pi_ref[j:j + 1, :] = idx
        s = jnp.where(iota == idx, -jnp.inf, s)


@jax.jit
def kernel(x, W_out, W_proc):
    B, S, R = x.shape
    n_tok = B * S
    xf = x.reshape(n_tok, R)
    grid = (n_tok // _BLK,)
    ow_t, pi_t = pl.pallas_call(
        _router_kernel,
        grid=grid,
        in_specs=[
            pl.BlockSpec((_BLK, _HALF), lambda i: (i, 0)),
            pl.BlockSpec((_BLK, _HALF), lambda i: (i, 1)),
            pl.BlockSpec((_N_OUT, R), lambda i: (0, 0)),
            pl.BlockSpec((_N_PROC, R), lambda i: (0, 0)),
        ],
        out_specs=[
            pl.BlockSpec((_N_OUT, _BLK), lambda i: (0, i)),
            pl.BlockSpec((_K, _BLK), lambda i: (0, i)),
        ],
        out_shape=[
            jax.ShapeDtypeStruct((_N_OUT, n_tok), jnp.float32),
            jax.ShapeDtypeStruct((_K, n_tok), jnp.int32),
        ],
        compiler_params=pltpu.CompilerParams(
            dimension_semantics=("arbitrary",),
        ),
    )(xf, xf, W_out, W_proc)
    ow = ow_t.T.reshape(B, S, _N_OUT)
    pi = pi_t.T.reshape(B, S, _K)
    return ow, pi
